# baseline (speedup 1.0000x reference)
import functools

import jax
import jax.numpy as jnp
from jax.experimental import pallas as pl
from jax.experimental.pallas import tpu as pltpu

DIM = 1024
HIDDEN = 2048
NUM_EXPERTS = 8
TOP_K = 2
TOKENS = 2048

CHUNK = 128
NCHUNK = TOKENS // CHUNK
B_ROWS = 256
NB = 24
P = NB * B_ROWS
HT = 512
NH = HIDDEN // HT


def _route_kernel(x_ref, wr_ref, i1_ref, i2_ref, g1_ref, g2_ref,
                  r1_ref, r2_ref, cnt_ref, carry_ref):
    c = pl.program_id(0)

    @pl.when(c == 0)
    def _():
        carry_ref[...] = jnp.zeros_like(carry_ref)

    x = x_ref[...]
    wr = wr_ref[...]
    logits = jax.lax.dot_general(
        x, wr, (((1,), (1,)), ((), ())),
        preferred_element_type=jnp.float32)

    e_iota = jax.lax.broadcasted_iota(jnp.int32, (CHUNK, NUM_EXPERTS), 1)
    m1 = jnp.max(logits, axis=1, keepdims=True)
    i1 = jnp.min(jnp.where(logits == m1, e_iota, NUM_EXPERTS), axis=1)
    oh1 = e_iota == i1[:, None]
    masked = jnp.where(oh1, -jnp.inf, logits)
    m2 = jnp.max(masked, axis=1, keepdims=True)
    i2 = jnp.min(jnp.where(masked == m2, e_iota, NUM_EXPERTS), axis=1)
    oh2 = e_iota == i2[:, None]

    g1 = 1.0 / (1.0 + jnp.exp(m2 - m1))
    g2 = 1.0 - g1

    c_oh = oh1.astype(jnp.float32) + oh2.astype(jnp.float32)

    ti = jax.lax.broadcasted_iota(jnp.int32, (CHUNK, CHUNK), 0)
    tj = jax.lax.broadcasted_iota(jnp.int32, (CHUNK, CHUNK), 1)
    tril = (ti > tj).astype(jnp.float32)
    carry = carry_ref[...]
    ranks = jax.lax.dot_general(
        tril, c_oh, (((1,), (0,)), ((), ())),
        preferred_element_type=jnp.float32) + carry

    r1 = jnp.sum(jnp.where(oh1, ranks, 0.0), axis=1)
    r2 = jnp.sum(jnp.where(oh2, ranks, 0.0), axis=1)

    new_carry = carry + jnp.sum(c_oh, axis=0, keepdims=True)
    carry_ref[...] = new_carry

    i1_ref[...] = i1.astype(jnp.int32)
    i2_ref[...] = i2.astype(jnp.int32)
    g1_ref[...] = g1[:, 0]
    g2_ref[...] = g2[:, 0]
    r1_ref[...] = r1.astype(jnp.int32)
    r2_ref[...] = r2.astype(jnp.int32)
    cnt_ref[...] = jnp.pad(new_carry, ((0, 0), (0, 8)))[0].astype(jnp.int32)


def _route(x, Wr):
    vec = lambda: pl.BlockSpec((CHUNK,), lambda c: (c,))
    return pl.pallas_call(
        _route_kernel,
        grid=(NCHUNK,),
        in_specs=[
            pl.BlockSpec((CHUNK, DIM), lambda c: (c, 0)),
            pl.BlockSpec((NUM_EXPERTS, DIM), lambda c: (0, 0)),
        ],
        out_specs=[vec(), vec(), vec(), vec(), vec(), vec(),
                   pl.BlockSpec((16,), lambda c: (0,))],
        out_shape=[
            jax.ShapeDtypeStruct((TOKENS,), jnp.int32),
            jax.ShapeDtypeStruct((TOKENS,), jnp.int32),
            jax.ShapeDtypeStruct((TOKENS,), jnp.float32),
            jax.ShapeDtypeStruct((TOKENS,), jnp.float32),
            jax.ShapeDtypeStruct((TOKENS,), jnp.int32),
            jax.ShapeDtypeStruct((TOKENS,), jnp.int32),
            jax.ShapeDtypeStruct((16,), jnp.int32),
        ],
        scratch_shapes=[pltpu.VMEM((1, NUM_EXPERTS), jnp.float32)],
    )(x, Wr)


def _ffn_kernel(g_ref, x_ref, w1_ref, w3_ref, w2_ref, out_ref):
    h = pl.program_id(1)
    x = x_ref[...]
    w1 = w1_ref[0]
    w3 = w3_ref[0]
    w2 = w2_ref[0]
    h1 = jax.lax.dot_general(x, w1, (((1,), (1,)), ((), ())),
                             preferred_element_type=jnp.float32)
    h3 = jax.lax.dot_general(x, w3, (((1,), (1,)), ((), ())),
                             preferred_element_type=jnp.float32)
    hh = (h1 * jax.nn.sigmoid(h1)) * h3
    y = jax.lax.dot_general(hh, w2, (((1,), (1,)), ((), ())),
                            preferred_element_type=jnp.float32)

    @pl.when(h == 0)
    def _():
        out_ref[...] = y

    @pl.when(h != 0)
    def _():
        out_ref[...] += y


def _ffn(g_blk, xs, W1, W3, W2):
    return pl.pallas_call(
        _ffn_kernel,
        grid_spec=pltpu.PrefetchScalarGridSpec(
            num_scalar_prefetch=1,
            grid=(NB, NH),
            in_specs=[
                pl.BlockSpec((B_ROWS, DIM), lambda b, h, g: (b, 0)),
                pl.BlockSpec((1, HT, DIM), lambda b, h, g: (g[b], h, 0)),
                pl.BlockSpec((1, HT, DIM), lambda b, h, g: (g[b], h, 0)),
                pl.BlockSpec((1, DIM, HT), lambda b, h, g: (g[b], 0, h)),
            ],
            out_specs=pl.BlockSpec((B_ROWS, DIM), lambda b, h, g: (b, 0)),
        ),
        out_shape=jax.ShapeDtypeStruct((P, DIM), jnp.float32),
    )(g_blk, xs, W1, W3, W2)


def kernel(x, Wr, W1, W2, W3):
    B, S, D = x.shape
    xf = x.reshape(-1, D)

    i1, i2, g1, g2, r1, r2, cnt = _route(xf, Wr)

    c = cnt[:NUM_EXPERTS]
    used = (c + B_ROWS - 1) // B_ROWS
    pc = used * B_ROWS
    off = jnp.cumsum(pc) - pc
    ends = jnp.cumsum(used)
    g_blk = jnp.minimum(
        jnp.sum((jnp.arange(NB)[:, None] >= ends[None, :]).astype(jnp.int32),
                axis=1), NUM_EXPERTS - 1).astype(jnp.int32)
    pos1 = off[i1] + r1
    pos2 = off[i2] + r2
    xs = jnp.zeros((P, D), xf.dtype).at[pos1].set(xf).at[pos2].set(xf)

    ys = _ffn(g_blk, xs, W1, W3, W2)

    out = g1[:, None] * ys[pos1] + g2[:, None] * ys[pos2]
    return out.reshape(B, S, D)


# speedup vs baseline: 1.3593x; 1.3593x over previous
import functools

import jax
import jax.numpy as jnp
from jax.experimental import pallas as pl
from jax.experimental.pallas import tpu as pltpu

DIM = 1024
HIDDEN = 2048
NUM_EXPERTS = 8
TOP_K = 2
TOKENS = 2048

CHUNK = 128
NCHUNK = TOKENS // CHUNK
B_ROWS = 256
NB = 24
P = NB * B_ROWS
HT = 512
NH = HIDDEN // HT


def _route_kernel(x_ref, wr_ref, i1_ref, i2_ref, g1_ref, g2_ref,
                  r1_ref, r2_ref, cnt_ref, carry_ref):
    c = pl.program_id(0)

    @pl.when(c == 0)
    def _():
        carry_ref[...] = jnp.zeros_like(carry_ref)

    x = x_ref[...]
    wr = wr_ref[...]
    logits = jax.lax.dot_general(
        x, wr, (((1,), (1,)), ((), ())),
        preferred_element_type=jnp.float32)

    e_iota = jax.lax.broadcasted_iota(jnp.int32, (CHUNK, NUM_EXPERTS), 1)
    m1 = jnp.max(logits, axis=1, keepdims=True)
    i1 = jnp.min(jnp.where(logits == m1, e_iota, NUM_EXPERTS), axis=1)
    oh1 = e_iota == i1[:, None]
    masked = jnp.where(oh1, -jnp.inf, logits)
    m2 = jnp.max(masked, axis=1, keepdims=True)
    i2 = jnp.min(jnp.where(masked == m2, e_iota, NUM_EXPERTS), axis=1)
    oh2 = e_iota == i2[:, None]

    g1 = 1.0 / (1.0 + jnp.exp(m2 - m1))
    g2 = 1.0 - g1

    c_oh = oh1.astype(jnp.float32) + oh2.astype(jnp.float32)

    ti = jax.lax.broadcasted_iota(jnp.int32, (CHUNK, CHUNK), 0)
    tj = jax.lax.broadcasted_iota(jnp.int32, (CHUNK, CHUNK), 1)
    tril = (ti > tj).astype(jnp.float32)
    carry = carry_ref[...]
    ranks = jax.lax.dot_general(
        tril, c_oh, (((1,), (0,)), ((), ())),
        preferred_element_type=jnp.float32) + carry

    r1 = jnp.sum(jnp.where(oh1, ranks, 0.0), axis=1)
    r2 = jnp.sum(jnp.where(oh2, ranks, 0.0), axis=1)

    new_carry = carry + jnp.sum(c_oh, axis=0, keepdims=True)
    carry_ref[...] = new_carry

    i1_ref[...] = i1.astype(jnp.int32)
    i2_ref[...] = i2.astype(jnp.int32)
    g1_ref[...] = g1[:, 0]
    g2_ref[...] = g2[:, 0]
    r1_ref[...] = r1.astype(jnp.int32)
    r2_ref[...] = r2.astype(jnp.int32)
    cnt_ref[...] = jnp.pad(new_carry, ((0, 0), (0, 8)))[0].astype(jnp.int32)


def _route(x, Wr):
    vec = lambda: pl.BlockSpec((CHUNK,), lambda c: (c,))
    return pl.pallas_call(
        _route_kernel,
        grid=(NCHUNK,),
        in_specs=[
            pl.BlockSpec((CHUNK, DIM), lambda c: (c, 0)),
            pl.BlockSpec((NUM_EXPERTS, DIM), lambda c: (0, 0)),
        ],
        out_specs=[vec(), vec(), vec(), vec(), vec(), vec(),
                   pl.BlockSpec((16,), lambda c: (0,))],
        out_shape=[
            jax.ShapeDtypeStruct((TOKENS,), jnp.int32),
            jax.ShapeDtypeStruct((TOKENS,), jnp.int32),
            jax.ShapeDtypeStruct((TOKENS,), jnp.float32),
            jax.ShapeDtypeStruct((TOKENS,), jnp.float32),
            jax.ShapeDtypeStruct((TOKENS,), jnp.int32),
            jax.ShapeDtypeStruct((TOKENS,), jnp.int32),
            jax.ShapeDtypeStruct((16,), jnp.int32),
        ],
        scratch_shapes=[pltpu.VMEM((1, NUM_EXPERTS), jnp.float32)],
    )(x, Wr)


def _ffn_kernel(g_ref, x_ref, w1_ref, w3_ref, w2_ref, out_ref):
    b = pl.program_id(0)

    @pl.when(b < g_ref[NB])
    def _():
        x = x_ref[...]
        w1 = w1_ref[0]
        w3 = w3_ref[0]
        w2 = w2_ref[0]
        h1 = jax.lax.dot_general(x, w1, (((1,), (1,)), ((), ())),
                                 preferred_element_type=jnp.float32)
        h3 = jax.lax.dot_general(x, w3, (((1,), (1,)), ((), ())),
                                 preferred_element_type=jnp.float32)
        hh = (h1 * jax.nn.sigmoid(h1)) * h3
        out_ref[...] = jax.lax.dot_general(
            hh, w2, (((1,), (1,)), ((), ())),
            preferred_element_type=jnp.float32)


def _ffn(g_blk, xs, W1, W3, W2):
    return pl.pallas_call(
        _ffn_kernel,
        grid_spec=pltpu.PrefetchScalarGridSpec(
            num_scalar_prefetch=1,
            grid=(NB,),
            in_specs=[
                pl.BlockSpec((B_ROWS, DIM), lambda b, g: (b, 0)),
                pl.BlockSpec((1, HIDDEN, DIM), lambda b, g: (g[b], 0, 0)),
                pl.BlockSpec((1, HIDDEN, DIM), lambda b, g: (g[b], 0, 0)),
                pl.BlockSpec((1, DIM, HIDDEN), lambda b, g: (g[b], 0, 0)),
            ],
            out_specs=pl.BlockSpec((B_ROWS, DIM), lambda b, g: (b, 0)),
        ),
        out_shape=jax.ShapeDtypeStruct((P, DIM), jnp.float32),
    )(g_blk, xs, W1, W3, W2)


def kernel(x, Wr, W1, W2, W3):
    B, S, D = x.shape
    xf = x.reshape(-1, D)

    i1, i2, g1, g2, r1, r2, cnt = _route(xf, Wr)

    c = cnt[:NUM_EXPERTS]
    used = (c + B_ROWS - 1) // B_ROWS
    pc = used * B_ROWS
    off = jnp.cumsum(pc) - pc
    ends = jnp.cumsum(used)
    nb_used = ends[NUM_EXPERTS - 1]
    g_blk = jnp.minimum(
        jnp.sum((jnp.arange(NB)[:, None] >= ends[None, :]).astype(jnp.int32),
                axis=1), NUM_EXPERTS - 1).astype(jnp.int32)
    last_e = jnp.max(jnp.where(used > 0, jnp.arange(NUM_EXPERTS), -1))
    g_blk = jnp.where(jnp.arange(NB) < nb_used, g_blk, last_e).astype(jnp.int32)
    g_blk = jnp.concatenate([g_blk, nb_used[None].astype(jnp.int32)])
    pos1 = off[i1] + r1
    pos2 = off[i2] + r2
    xs = jnp.zeros((P, D), xf.dtype).at[pos1].set(xf).at[pos2].set(xf)

    ys = _ffn(g_blk, xs, W1, W3, W2)

    out = g1[:, None] * ys[pos1] + g2[:, None] * ys[pos2]
    return out.reshape(B, S, D)


# speedup vs baseline: 1.5983x; 1.1758x over previous
import functools

import jax
import jax.numpy as jnp
from jax import lax
from jax.experimental import pallas as pl
from jax.experimental.pallas import tpu as pltpu
from jax.experimental.pallas import tpu_sc as plsc

DIM = 1024
HIDDEN = 2048
NUM_EXPERTS = 8
TOP_K = 2
TOKENS = 2048

CHUNK = 128
NCHUNK = TOKENS // CHUNK
B_ROWS = 256
NB = 24
P = NB * B_ROWS
HT = 512
NH = HIDDEN // HT


def _route_kernel(x_ref, wr_ref, i1_ref, i2_ref, g1_ref, g2_ref,
                  r1_ref, r2_ref, cnt_ref, carry_ref):
    c = pl.program_id(0)

    @pl.when(c == 0)
    def _():
        carry_ref[...] = jnp.zeros_like(carry_ref)

    x = x_ref[...]
    wr = wr_ref[...]
    logits = jax.lax.dot_general(
        x, wr, (((1,), (1,)), ((), ())),
        preferred_element_type=jnp.float32)

    e_iota = jax.lax.broadcasted_iota(jnp.int32, (CHUNK, NUM_EXPERTS), 1)
    m1 = jnp.max(logits, axis=1, keepdims=True)
    i1 = jnp.min(jnp.where(logits == m1, e_iota, NUM_EXPERTS), axis=1)
    oh1 = e_iota == i1[:, None]
    masked = jnp.where(oh1, -jnp.inf, logits)
    m2 = jnp.max(masked, axis=1, keepdims=True)
    i2 = jnp.min(jnp.where(masked == m2, e_iota, NUM_EXPERTS), axis=1)
    oh2 = e_iota == i2[:, None]

    g1 = 1.0 / (1.0 + jnp.exp(m2 - m1))
    g2 = 1.0 - g1

    c_oh = oh1.astype(jnp.float32) + oh2.astype(jnp.float32)

    ti = jax.lax.broadcasted_iota(jnp.int32, (CHUNK, CHUNK), 0)
    tj = jax.lax.broadcasted_iota(jnp.int32, (CHUNK, CHUNK), 1)
    tril = (ti > tj).astype(jnp.float32)
    carry = carry_ref[...]
    ranks = jax.lax.dot_general(
        tril, c_oh, (((1,), (0,)), ((), ())),
        preferred_element_type=jnp.float32) + carry

    r1 = jnp.sum(jnp.where(oh1, ranks, 0.0), axis=1)
    r2 = jnp.sum(jnp.where(oh2, ranks, 0.0), axis=1)

    new_carry = carry + jnp.sum(c_oh, axis=0, keepdims=True)
    carry_ref[...] = new_carry

    i1_ref[...] = i1.astype(jnp.int32)
    i2_ref[...] = i2.astype(jnp.int32)
    g1_ref[...] = g1[:, 0]
    g2_ref[...] = g2[:, 0]
    r1_ref[...] = r1.astype(jnp.int32)
    r2_ref[...] = r2.astype(jnp.int32)
    cnt_ref[...] = jnp.pad(new_carry, ((0, 0), (0, 8)))[0].astype(jnp.int32)


def _route(x, Wr):
    vec = lambda: pl.BlockSpec((CHUNK,), lambda c: (c,))
    return pl.pallas_call(
        _route_kernel,
        grid=(NCHUNK,),
        in_specs=[
            pl.BlockSpec((CHUNK, DIM), lambda c: (c, 0)),
            pl.BlockSpec((NUM_EXPERTS, DIM), lambda c: (0, 0)),
        ],
        out_specs=[vec(), vec(), vec(), vec(), vec(), vec(),
                   pl.BlockSpec((16,), lambda c: (0,))],
        out_shape=[
            jax.ShapeDtypeStruct((TOKENS,), jnp.int32),
            jax.ShapeDtypeStruct((TOKENS,), jnp.int32),
            jax.ShapeDtypeStruct((TOKENS,), jnp.float32),
            jax.ShapeDtypeStruct((TOKENS,), jnp.float32),
            jax.ShapeDtypeStruct((TOKENS,), jnp.int32),
            jax.ShapeDtypeStruct((TOKENS,), jnp.int32),
            jax.ShapeDtypeStruct((16,), jnp.int32),
        ],
        scratch_shapes=[pltpu.VMEM((1, NUM_EXPERTS), jnp.float32)],
    )(x, Wr)


def _ffn_kernel(g_ref, x_ref, w1_ref, w3_ref, w2_ref, out_ref):
    b = pl.program_id(0)

    @pl.when(b < g_ref[NB])
    def _():
        x = x_ref[...]
        w1 = w1_ref[0]
        w3 = w3_ref[0]
        w2 = w2_ref[0]
        h1 = jax.lax.dot_general(x, w1, (((1,), (1,)), ((), ())),
                                 preferred_element_type=jnp.float32)
        h3 = jax.lax.dot_general(x, w3, (((1,), (1,)), ((), ())),
                                 preferred_element_type=jnp.float32)
        hh = (h1 * jax.nn.sigmoid(h1)) * h3
        out_ref[...] = jax.lax.dot_general(
            hh, w2, (((1,), (1,)), ((), ())),
            preferred_element_type=jnp.float32)


def _ffn(g_blk, xs, W1, W3, W2):
    return pl.pallas_call(
        _ffn_kernel,
        grid_spec=pltpu.PrefetchScalarGridSpec(
            num_scalar_prefetch=1,
            grid=(NB,),
            in_specs=[
                pl.BlockSpec((B_ROWS, DIM), lambda b, g: (b, 0)),
                pl.BlockSpec((1, HIDDEN, DIM), lambda b, g: (g[b], 0, 0)),
                pl.BlockSpec((1, HIDDEN, DIM), lambda b, g: (g[b], 0, 0)),
                pl.BlockSpec((1, DIM, HIDDEN), lambda b, g: (g[b], 0, 0)),
            ],
            out_specs=pl.BlockSpec((B_ROWS, DIM), lambda b, g: (b, 0)),
        ),
        out_shape=jax.ShapeDtypeStruct((P, DIM), jnp.float32),
    )(g_blk, xs, W1, W3, W2)


SC_CORES = 2
SC_SUBCORES = 16
NW = SC_CORES * SC_SUBCORES
TW = TOKENS // NW
HALF = TW // 2


def _sc_wid():
    return lax.axis_index("s") * SC_CORES + lax.axis_index("c")


def _dispatch_sc(x, i1, i2, r1, r2, cnt):
    mesh = plsc.VectorSubcoreMesh(core_axis_name="c", subcore_axis_name="s", num_cores=SC_CORES, num_subcores=SC_SUBCORES)

    @functools.partial(
        pl.kernel,
        out_type=(
            jax.ShapeDtypeStruct((P, DIM), jnp.float32),
            jax.ShapeDtypeStruct((TOKENS,), jnp.int32),
            jax.ShapeDtypeStruct((TOKENS,), jnp.int32),
            jax.ShapeDtypeStruct((32,), jnp.int32),
        ),
        mesh=mesh,
        compiler_params=pltpu.CompilerParams(needs_layout_passes=False),
        scratch_types=[
            pltpu.VMEM((16,), jnp.int32),
            pltpu.VMEM((16,), jnp.int32),
            pltpu.VMEM((16,), jnp.int32),
            pltpu.VMEM((TW,), jnp.int32),
            pltpu.VMEM((TW,), jnp.int32),
            pltpu.VMEM((TW,), jnp.int32),
            pltpu.VMEM((TW,), jnp.int32),
            pltpu.VMEM((TW, DIM), jnp.float32),
            pltpu.VMEM((32,), jnp.int32),
            pltpu.SemaphoreType.DMA,
        ],
    )
    def k(x_hbm, i1_hbm, i2_hbm, r1_hbm, r2_hbm, cnt_hbm,
          xs_hbm, pos1_hbm, pos2_hbm, gout_hbm,
          cnt_v, off_v, ends_v, i_v, r_v, pos1_v, pos2_v, rows_v, gout_v,
          sem):
        wid = _sc_wid()
        base = wid * TW
        pltpu.sync_copy(cnt_hbm, cnt_v)
        c = cnt_v[...]
        used = (c + (B_ROWS - 1)) >> 8
        ends = plsc.cumsum(used)
        off = (ends - used) * B_ROWS
        off_v[...] = off
        ends_v[...] = ends

        for (ih, rh, pv) in ((i1_hbm, r1_hbm, pos1_v),
                             (i2_hbm, r2_hbm, pos2_v)):
            pltpu.sync_copy(ih.at[pl.ds(base, TW)], i_v)
            pltpu.sync_copy(rh.at[pl.ds(base, TW)], r_v)
            for j in range(TW // 16):
                sl = pl.ds(j * 16, 16)
                idx = i_v[sl]
                pv[sl] = plsc.load_gather(off_v, [idx]) + r_v[sl]

        pltpu.sync_copy(x_hbm.at[pl.ds(base, TW)], rows_v)
        pltpu.async_copy(rows_v, xs_hbm.at[pos1_v], sem).wait()
        pltpu.async_copy(rows_v, xs_hbm.at[pos2_v], sem).wait()
        pltpu.sync_copy(pos1_v, pos1_hbm.at[pl.ds(base, TW)])
        pltpu.sync_copy(pos2_v, pos2_hbm.at[pl.ds(base, TW)])

        @pl.when(wid == 0)
        def _():
            iota = lax.iota(jnp.int32, 16)
            ends2 = ends_v[...]
            used2 = cnt_v[...]
            usedb = (used2 + (B_ROWS - 1)) >> 8
            nb_used = lax.reduce_max(ends2, (0,))
            last_e = lax.reduce_max(
                jnp.where(usedb > 0, iota, -1), (0,))
            gb0 = jnp.zeros((16,), jnp.int32)
            gb1 = jnp.zeros((16,), jnp.int32)
            b0 = iota
            b1 = iota + 16
            for e in range(NUM_EXPERTS):
                ends_e = lax.reduce_max(
                    jnp.where(iota == e, ends2, -1), (0,))
                gb0 += (b0 >= ends_e).astype(jnp.int32)
                gb1 += (b1 >= ends_e).astype(jnp.int32)
            gb0 = jnp.where(b0 < nb_used, jnp.minimum(gb0, NUM_EXPERTS - 1),
                            last_e)
            gb1 = jnp.where(b1 < nb_used, jnp.minimum(gb1, NUM_EXPERTS - 1),
                            last_e)
            gb1 = jnp.where(iota == 8, nb_used, gb1)
            gout_v[pl.ds(0, 16)] = gb0
            gout_v[pl.ds(16, 16)] = gb1
            pltpu.sync_copy(gout_v, gout_hbm)

    return k(x, i1, i2, r1, r2, cnt)


def _combine_sc(ys, pos1, pos2, g1, g2):
    mesh = plsc.VectorSubcoreMesh(core_axis_name="c", subcore_axis_name="s", num_cores=SC_CORES, num_subcores=SC_SUBCORES)

    @functools.partial(
        pl.kernel,
        out_type=jax.ShapeDtypeStruct((TOKENS, DIM), jnp.float32),
        mesh=mesh,
        compiler_params=pltpu.CompilerParams(needs_layout_passes=False),
        scratch_types=[
            [pltpu.VMEM((HALF,), jnp.int32) for _ in range(2)],
            [pltpu.VMEM((HALF,), jnp.int32) for _ in range(2)],
            pltpu.VMEM((TW,), jnp.float32),
            pltpu.VMEM((TW,), jnp.float32),
            pltpu.VMEM((HALF, DIM), jnp.float32),
            pltpu.VMEM((HALF, DIM), jnp.float32),
            pltpu.SemaphoreType.DMA,
            pltpu.SemaphoreType.DMA,
        ],
    )
    def k(ys_hbm, pos1_hbm, pos2_hbm, g1_hbm, g2_hbm, out_hbm,
          pos1_vs, pos2_vs, g1_v, g2_v, rows1_v, rows2_v, sem1, sem2):
        wid = _sc_wid()
        base = wid * TW
        pltpu.sync_copy(g1_hbm.at[pl.ds(base, TW)], g1_v)
        pltpu.sync_copy(g2_hbm.at[pl.ds(base, TW)], g2_v)
        for h in range(2):
            pltpu.sync_copy(pos1_hbm.at[pl.ds(base + h * HALF, HALF)],
                            pos1_vs[h])
            pltpu.sync_copy(pos2_hbm.at[pl.ds(base + h * HALF, HALF)],
                            pos2_vs[h])
        for h in range(2):
            cp1 = pltpu.async_copy(ys_hbm.at[pos1_vs[h]], rows1_v, sem1)
            cp2 = pltpu.async_copy(ys_hbm.at[pos2_vs[h]], rows2_v, sem2)
            cp1.wait()
            cp2.wait()

            def body(t, _):
                tidx = jnp.zeros((16,), jnp.int32) + (h * HALF + t)
                ga = plsc.load_gather(g1_v, [tidx])
                gb = plsc.load_gather(g2_v, [tidx])

                def inner(cc, _):
                    sl = pl.ds(cc * 16, 16)
                    rows1_v[t, sl] = rows1_v[t, sl] * ga + rows2_v[t, sl] * gb
                    return 0

                return lax.fori_loop(0, DIM // 16, inner, 0, unroll=8)

            lax.fori_loop(0, HALF, body, 0)
            pltpu.sync_copy(rows1_v,
                            out_hbm.at[pl.ds(base + h * HALF, HALF)])

    return k(ys, pos1, pos2, g1, g2)


def kernel(x, Wr, W1, W2, W3):
    B, S, D = x.shape
    xf = x.reshape(-1, D)

    i1, i2, g1, g2, r1, r2, cnt = _route(xf, Wr)
    xs, pos1, pos2, gout = _dispatch_sc(xf, i1, i2, r1, r2, cnt)
    ys = _ffn(gout[:NB + 1], xs, W1, W3, W2)
    out = _combine_sc(ys, pos1, pos2, g1, g2)
    return out.reshape(B, S, D)


# speedup vs baseline: 1.7456x; 1.0922x over previous
import functools

import jax
import jax.numpy as jnp
from jax import lax
from jax.experimental import pallas as pl
from jax.experimental.pallas import tpu as pltpu
from jax.experimental.pallas import tpu_sc as plsc

DIM = 1024
HIDDEN = 2048
NUM_EXPERTS = 8
TOP_K = 2
TOKENS = 2048

CHUNK = 128
NCHUNK = TOKENS // CHUNK
B_ROWS = 256
NB = 24
P = NB * B_ROWS
HT = 512
NH = HIDDEN // HT


def _route_kernel(x_ref, wr_ref, i1_ref, i2_ref, g1_ref, g2_ref,
                  r1_ref, r2_ref, cnt_ref, carry_ref):
    c = pl.program_id(0)

    @pl.when(c == 0)
    def _():
        carry_ref[...] = jnp.zeros_like(carry_ref)

    x = x_ref[...]
    wr = wr_ref[...]
    logits = jax.lax.dot_general(
        x, wr, (((1,), (1,)), ((), ())),
        preferred_element_type=jnp.float32)

    e_iota = jax.lax.broadcasted_iota(jnp.int32, (CHUNK, NUM_EXPERTS), 1)
    m1 = jnp.max(logits, axis=1, keepdims=True)
    i1 = jnp.min(jnp.where(logits == m1, e_iota, NUM_EXPERTS), axis=1)
    oh1 = e_iota == i1[:, None]
    masked = jnp.where(oh1, -jnp.inf, logits)
    m2 = jnp.max(masked, axis=1, keepdims=True)
    i2 = jnp.min(jnp.where(masked == m2, e_iota, NUM_EXPERTS), axis=1)
    oh2 = e_iota == i2[:, None]

    g1 = 1.0 / (1.0 + jnp.exp(m2 - m1))
    g2 = 1.0 - g1

    c_oh = oh1.astype(jnp.float32) + oh2.astype(jnp.float32)

    ti = jax.lax.broadcasted_iota(jnp.int32, (CHUNK, CHUNK), 0)
    tj = jax.lax.broadcasted_iota(jnp.int32, (CHUNK, CHUNK), 1)
    tril = (ti > tj).astype(jnp.float32)
    carry = carry_ref[...]
    ranks = jax.lax.dot_general(
        tril, c_oh, (((1,), (0,)), ((), ())),
        preferred_element_type=jnp.float32) + carry

    r1 = jnp.sum(jnp.where(oh1, ranks, 0.0), axis=1)
    r2 = jnp.sum(jnp.where(oh2, ranks, 0.0), axis=1)

    new_carry = carry + jnp.sum(c_oh, axis=0, keepdims=True)
    carry_ref[...] = new_carry

    i1_ref[...] = i1.astype(jnp.int32)
    i2_ref[...] = i2.astype(jnp.int32)
    g1_ref[...] = g1[:, 0]
    g2_ref[...] = g2[:, 0]
    r1_ref[...] = r1.astype(jnp.int32)
    r2_ref[...] = r2.astype(jnp.int32)
    cnt_ref[...] = jnp.pad(new_carry, ((0, 0), (0, 8)))[0].astype(jnp.int32)


def _route(x, Wr):
    vec = lambda: pl.BlockSpec((CHUNK,), lambda c: (c,))
    return pl.pallas_call(
        _route_kernel,
        grid=(NCHUNK,),
        in_specs=[
            pl.BlockSpec((CHUNK, DIM), lambda c: (c, 0)),
            pl.BlockSpec((NUM_EXPERTS, DIM), lambda c: (0, 0)),
        ],
        out_specs=[vec(), vec(), vec(), vec(), vec(), vec(),
                   pl.BlockSpec((16,), lambda c: (0,))],
        out_shape=[
            jax.ShapeDtypeStruct((TOKENS,), jnp.int32),
            jax.ShapeDtypeStruct((TOKENS,), jnp.int32),
            jax.ShapeDtypeStruct((TOKENS,), jnp.float32),
            jax.ShapeDtypeStruct((TOKENS,), jnp.float32),
            jax.ShapeDtypeStruct((TOKENS,), jnp.int32),
            jax.ShapeDtypeStruct((TOKENS,), jnp.int32),
            jax.ShapeDtypeStruct((16,), jnp.int32),
        ],
        scratch_shapes=[pltpu.VMEM((1, NUM_EXPERTS), jnp.float32)],
    )(x, Wr)


def _ffn_kernel(g_ref, x_ref, w1_ref, w3_ref, w2_ref, out_ref):
    b = pl.program_id(0)

    @pl.when(b < g_ref[NB])
    def _():
        x = x_ref[...]
        w1 = w1_ref[0]
        w3 = w3_ref[0]
        w2 = w2_ref[0]
        h1 = jax.lax.dot_general(x, w1, (((1,), (1,)), ((), ())),
                                 preferred_element_type=jnp.float32)
        h3 = jax.lax.dot_general(x, w3, (((1,), (1,)), ((), ())),
                                 preferred_element_type=jnp.float32)
        hh = (h1 * jax.nn.sigmoid(h1)) * h3
        out_ref[...] = jax.lax.dot_general(
            hh, w2, (((1,), (1,)), ((), ())),
            preferred_element_type=jnp.float32)


def _ffn(g_blk, xs, W1, W3, W2):
    return pl.pallas_call(
        _ffn_kernel,
        grid_spec=pltpu.PrefetchScalarGridSpec(
            num_scalar_prefetch=1,
            grid=(NB,),
            in_specs=[
                pl.BlockSpec((B_ROWS, DIM), lambda b, g: (b, 0)),
                pl.BlockSpec((1, HIDDEN, DIM), lambda b, g: (g[b], 0, 0)),
                pl.BlockSpec((1, HIDDEN, DIM), lambda b, g: (g[b], 0, 0)),
                pl.BlockSpec((1, DIM, HIDDEN), lambda b, g: (g[b], 0, 0)),
            ],
            out_specs=pl.BlockSpec((B_ROWS, DIM), lambda b, g: (b, 0)),
        ),
        out_shape=jax.ShapeDtypeStruct((P, DIM), jnp.float32),
    )(g_blk, xs, W1, W3, W2)


SC_CORES = 2
SC_SUBCORES = 16
NW = SC_CORES * SC_SUBCORES
TW = TOKENS // NW
HALF = TW // 2


def _sc_wid():
    return lax.axis_index("s") * SC_CORES + lax.axis_index("c")


def _dispatch_sc(x, i1, i2, r1, r2, cnt):
    mesh = plsc.VectorSubcoreMesh(core_axis_name="c", subcore_axis_name="s", num_cores=SC_CORES, num_subcores=SC_SUBCORES)

    @functools.partial(
        pl.kernel,
        out_type=(
            jax.ShapeDtypeStruct((P, DIM), jnp.float32),
            jax.ShapeDtypeStruct((TOKENS,), jnp.int32),
            jax.ShapeDtypeStruct((TOKENS,), jnp.int32),
            jax.ShapeDtypeStruct((32,), jnp.int32),
        ),
        mesh=mesh,
        compiler_params=pltpu.CompilerParams(needs_layout_passes=False),
        scratch_types=[
            pltpu.VMEM((16,), jnp.int32),
            pltpu.VMEM((16,), jnp.int32),
            pltpu.VMEM((16,), jnp.int32),
            pltpu.VMEM((TW,), jnp.int32),
            pltpu.VMEM((TW,), jnp.int32),
            pltpu.VMEM((TW,), jnp.int32),
            pltpu.VMEM((TW,), jnp.int32),
            pltpu.VMEM((TW, DIM), jnp.float32),
            pltpu.VMEM((32,), jnp.int32),
            pltpu.SemaphoreType.DMA,
        ],
    )
    def k(x_hbm, i1_hbm, i2_hbm, r1_hbm, r2_hbm, cnt_hbm,
          xs_hbm, pos1_hbm, pos2_hbm, gout_hbm,
          cnt_v, off_v, ends_v, i_v, r_v, pos1_v, pos2_v, rows_v, gout_v,
          sem):
        wid = _sc_wid()
        base = wid * TW
        pltpu.sync_copy(cnt_hbm, cnt_v)
        c = cnt_v[...]
        used = (c + (B_ROWS - 1)) >> 8
        ends = plsc.cumsum(used)
        off = (ends - used) * B_ROWS
        off_v[...] = off
        ends_v[...] = ends

        for (ih, rh, pv) in ((i1_hbm, r1_hbm, pos1_v),
                             (i2_hbm, r2_hbm, pos2_v)):
            pltpu.sync_copy(ih.at[pl.ds(base, TW)], i_v)
            pltpu.sync_copy(rh.at[pl.ds(base, TW)], r_v)
            for j in range(TW // 16):
                sl = pl.ds(j * 16, 16)
                idx = i_v[sl]
                pv[sl] = plsc.load_gather(off_v, [idx]) + r_v[sl]

        pltpu.sync_copy(x_hbm.at[pl.ds(base, TW)], rows_v)
        pltpu.async_copy(rows_v, xs_hbm.at[pos1_v], sem).wait()
        pltpu.async_copy(rows_v, xs_hbm.at[pos2_v], sem).wait()
        pltpu.sync_copy(pos1_v, pos1_hbm.at[pl.ds(base, TW)])
        pltpu.sync_copy(pos2_v, pos2_hbm.at[pl.ds(base, TW)])

        @pl.when(wid == 0)
        def _():
            iota = lax.iota(jnp.int32, 16)
            ends2 = ends_v[...]
            used2 = cnt_v[...]
            usedb = (used2 + (B_ROWS - 1)) >> 8
            nb_used = lax.reduce_max(ends2, (0,))
            last_e = lax.reduce_max(
                jnp.where(usedb > 0, iota, -1), (0,))
            gb0 = jnp.zeros((16,), jnp.int32)
            gb1 = jnp.zeros((16,), jnp.int32)
            b0 = iota
            b1 = iota + 16
            for e in range(NUM_EXPERTS):
                ends_e = lax.reduce_max(
                    jnp.where(iota == e, ends2, -1), (0,))
                gb0 += (b0 >= ends_e).astype(jnp.int32)
                gb1 += (b1 >= ends_e).astype(jnp.int32)
            gb0 = jnp.where(b0 < nb_used, jnp.minimum(gb0, NUM_EXPERTS - 1),
                            last_e)
            gb1 = jnp.where(b1 < nb_used, jnp.minimum(gb1, NUM_EXPERTS - 1),
                            last_e)
            gb1 = jnp.where(iota == 8, nb_used, gb1)
            gout_v[pl.ds(0, 16)] = gb0
            gout_v[pl.ds(16, 16)] = gb1
            pltpu.sync_copy(gout_v, gout_hbm)

    return k(x, i1, i2, r1, r2, cnt)


def _combine_sc(ys, pos1, pos2, g1, g2):
    mesh = plsc.VectorSubcoreMesh(core_axis_name="c", subcore_axis_name="s", num_cores=SC_CORES, num_subcores=SC_SUBCORES)

    @functools.partial(
        pl.kernel,
        out_type=jax.ShapeDtypeStruct((TOKENS, DIM), jnp.float32),
        mesh=mesh,
        compiler_params=pltpu.CompilerParams(needs_layout_passes=False),
        scratch_types=[
            [pltpu.VMEM((HALF,), jnp.int32) for _ in range(2)],
            [pltpu.VMEM((HALF,), jnp.int32) for _ in range(2)],
            pltpu.VMEM((TW,), jnp.float32),
            pltpu.VMEM((TW,), jnp.float32),
            pltpu.VMEM((HALF, DIM), jnp.float32),
            pltpu.VMEM((HALF, DIM), jnp.float32),
            pltpu.SemaphoreType.DMA,
            pltpu.SemaphoreType.DMA,
        ],
    )
    def k(ys_hbm, pos1_hbm, pos2_hbm, g1_hbm, g2_hbm, out_hbm,
          pos1_vs, pos2_vs, g1_v, g2_v, rows1_v, rows2_v, sem1, sem2):
        wid = _sc_wid()
        base = wid * TW
        pltpu.sync_copy(g1_hbm.at[pl.ds(base, TW)], g1_v)
        pltpu.sync_copy(g2_hbm.at[pl.ds(base, TW)], g2_v)
        for h in range(2):
            pltpu.sync_copy(pos1_hbm.at[pl.ds(base + h * HALF, HALF)],
                            pos1_vs[h])
            pltpu.sync_copy(pos2_hbm.at[pl.ds(base + h * HALF, HALF)],
                            pos2_vs[h])
        for h in range(2):
            cp1 = pltpu.async_copy(ys_hbm.at[pos1_vs[h]], rows1_v, sem1)
            cp2 = pltpu.async_copy(ys_hbm.at[pos2_vs[h]], rows2_v, sem2)
            cp1.wait()
            cp2.wait()

            @plsc.parallel_loop(0, HALF)
            def _(t):
                tidx = jnp.zeros((16,), jnp.int32) + (h * HALF + t)
                ga = plsc.load_gather(g1_v, [tidx])
                gb = plsc.load_gather(g2_v, [tidx])

                @plsc.parallel_loop(0, DIM // 16, unroll=8)
                def _(cc):
                    sl = pl.ds(cc * 16, 16)
                    rows1_v[t, sl] = rows1_v[t, sl] * ga + rows2_v[t, sl] * gb
            pltpu.sync_copy(rows1_v,
                            out_hbm.at[pl.ds(base + h * HALF, HALF)])

    return k(ys, pos1, pos2, g1, g2)


def kernel(x, Wr, W1, W2, W3):
    B, S, D = x.shape
    xf = x.reshape(-1, D)

    i1, i2, g1, g2, r1, r2, cnt = _route(xf, Wr)
    xs, pos1, pos2, gout = _dispatch_sc(xf, i1, i2, r1, r2, cnt)
    ys = _ffn(gout, xs, W1, W3, W2)
    out = _combine_sc(ys, pos1, pos2, g1, g2)
    return out.reshape(B, S, D)


# speedup vs baseline: 1.7957x; 1.0287x over previous
import functools

import jax
import jax.numpy as jnp
from jax import lax
from jax.experimental import pallas as pl
from jax.experimental.pallas import tpu as pltpu
from jax.experimental.pallas import tpu_sc as plsc

DIM = 1024
HIDDEN = 2048
NUM_EXPERTS = 8
TOP_K = 2
TOKENS = 2048

CHUNK = 256
NCHUNK = TOKENS // CHUNK
B_ROWS = 256
NB = 24
P = NB * B_ROWS
HT = 512
NH = HIDDEN // HT


def _route_kernel(x_ref, wr_ref, i1_ref, i2_ref, g1_ref, g2_ref,
                  r1_ref, r2_ref, cnt_ref, carry_ref):
    c = pl.program_id(0)

    @pl.when(c == 0)
    def _():
        carry_ref[...] = jnp.zeros_like(carry_ref)

    x = x_ref[...]
    wr = wr_ref[...]
    logits = jax.lax.dot_general(
        x, wr, (((1,), (1,)), ((), ())),
        preferred_element_type=jnp.float32)

    e_iota = jax.lax.broadcasted_iota(jnp.int32, (CHUNK, NUM_EXPERTS), 1)
    m1 = jnp.max(logits, axis=1, keepdims=True)
    i1 = jnp.min(jnp.where(logits == m1, e_iota, NUM_EXPERTS), axis=1)
    oh1 = e_iota == i1[:, None]
    masked = jnp.where(oh1, -jnp.inf, logits)
    m2 = jnp.max(masked, axis=1, keepdims=True)
    i2 = jnp.min(jnp.where(masked == m2, e_iota, NUM_EXPERTS), axis=1)
    oh2 = e_iota == i2[:, None]

    g1 = 1.0 / (1.0 + jnp.exp(m2 - m1))
    g2 = 1.0 - g1

    c_oh = oh1.astype(jnp.float32) + oh2.astype(jnp.float32)

    ti = jax.lax.broadcasted_iota(jnp.int32, (CHUNK, CHUNK), 0)
    tj = jax.lax.broadcasted_iota(jnp.int32, (CHUNK, CHUNK), 1)
    tril = (ti > tj).astype(jnp.float32)
    carry = carry_ref[...]
    ranks = jax.lax.dot_general(
        tril, c_oh, (((1,), (0,)), ((), ())),
        preferred_element_type=jnp.float32) + carry

    r1 = jnp.sum(jnp.where(oh1, ranks, 0.0), axis=1)
    r2 = jnp.sum(jnp.where(oh2, ranks, 0.0), axis=1)

    new_carry = carry + jnp.sum(c_oh, axis=0, keepdims=True)
    carry_ref[...] = new_carry

    i1_ref[...] = i1.astype(jnp.int32)
    i2_ref[...] = i2.astype(jnp.int32)
    g1_ref[...] = g1[:, 0]
    g2_ref[...] = g2[:, 0]
    r1_ref[...] = r1.astype(jnp.int32)
    r2_ref[...] = r2.astype(jnp.int32)
    cnt_ref[...] = jnp.pad(new_carry, ((0, 0), (0, 8)))[0].astype(jnp.int32)


def _route(x, Wr):
    vec = lambda: pl.BlockSpec((CHUNK,), lambda c: (c,))
    return pl.pallas_call(
        _route_kernel,
        grid=(NCHUNK,),
        in_specs=[
            pl.BlockSpec((CHUNK, DIM), lambda c: (c, 0)),
            pl.BlockSpec((NUM_EXPERTS, DIM), lambda c: (0, 0)),
        ],
        out_specs=[vec(), vec(), vec(), vec(), vec(), vec(),
                   pl.BlockSpec((16,), lambda c: (0,))],
        out_shape=[
            jax.ShapeDtypeStruct((TOKENS,), jnp.int32),
            jax.ShapeDtypeStruct((TOKENS,), jnp.int32),
            jax.ShapeDtypeStruct((TOKENS,), jnp.float32),
            jax.ShapeDtypeStruct((TOKENS,), jnp.float32),
            jax.ShapeDtypeStruct((TOKENS,), jnp.int32),
            jax.ShapeDtypeStruct((TOKENS,), jnp.int32),
            jax.ShapeDtypeStruct((16,), jnp.int32),
        ],
        scratch_shapes=[pltpu.VMEM((1, NUM_EXPERTS), jnp.float32)],
    )(x, Wr)


def _ffn_kernel(g_ref, x_ref, w1_ref, w3_ref, w2_ref, out_ref):
    b = pl.program_id(0)

    @pl.when(b < g_ref[NB])
    def _():
        x = x_ref[...]
        w1 = w1_ref[0]
        w3 = w3_ref[0]
        w2 = w2_ref[0]
        h1 = jax.lax.dot_general(x, w1, (((1,), (1,)), ((), ())),
                                 preferred_element_type=jnp.float32)
        h3 = jax.lax.dot_general(x, w3, (((1,), (1,)), ((), ())),
                                 preferred_element_type=jnp.float32)
        hh = (h1 * jax.nn.sigmoid(h1)) * h3
        out_ref[...] = jax.lax.dot_general(
            hh, w2, (((1,), (1,)), ((), ())),
            preferred_element_type=jnp.float32)


def _ffn(g_blk, xs, W1, W3, W2):
    return pl.pallas_call(
        _ffn_kernel,
        grid_spec=pltpu.PrefetchScalarGridSpec(
            num_scalar_prefetch=1,
            grid=(NB,),
            in_specs=[
                pl.BlockSpec((B_ROWS, DIM), lambda b, g: (b, 0)),
                pl.BlockSpec((1, HIDDEN, DIM), lambda b, g: (g[b], 0, 0)),
                pl.BlockSpec((1, HIDDEN, DIM), lambda b, g: (g[b], 0, 0)),
                pl.BlockSpec((1, DIM, HIDDEN), lambda b, g: (g[b], 0, 0)),
            ],
            out_specs=pl.BlockSpec((B_ROWS, DIM), lambda b, g: (b, 0)),
        ),
        out_shape=jax.ShapeDtypeStruct((P, DIM), jnp.float32),
    )(g_blk, xs, W1, W3, W2)


SC_CORES = 2
SC_SUBCORES = 16
NW = SC_CORES * SC_SUBCORES
TW = TOKENS // NW
HALF = TW // 2


def _sc_wid():
    return lax.axis_index("s") * SC_CORES + lax.axis_index("c")


def _dispatch_sc(x, i1, i2, r1, r2, cnt):
    mesh = plsc.VectorSubcoreMesh(core_axis_name="c", subcore_axis_name="s", num_cores=SC_CORES, num_subcores=SC_SUBCORES)

    @functools.partial(
        pl.kernel,
        out_type=(
            jax.ShapeDtypeStruct((P, DIM), jnp.float32),
            jax.ShapeDtypeStruct((TOKENS,), jnp.int32),
            jax.ShapeDtypeStruct((TOKENS,), jnp.int32),
            jax.ShapeDtypeStruct((32,), jnp.int32),
        ),
        mesh=mesh,
        compiler_params=pltpu.CompilerParams(needs_layout_passes=False),
        scratch_types=[
            pltpu.VMEM((16,), jnp.int32),
            pltpu.VMEM((16,), jnp.int32),
            pltpu.VMEM((16,), jnp.int32),
            pltpu.VMEM((TW,), jnp.int32),
            pltpu.VMEM((TW,), jnp.int32),
            pltpu.VMEM((TW,), jnp.int32),
            pltpu.VMEM((TW,), jnp.int32),
            pltpu.VMEM((TW, DIM), jnp.float32),
            pltpu.VMEM((32,), jnp.int32),
            pltpu.SemaphoreType.DMA,
        ],
    )
    def k(x_hbm, i1_hbm, i2_hbm, r1_hbm, r2_hbm, cnt_hbm,
          xs_hbm, pos1_hbm, pos2_hbm, gout_hbm,
          cnt_v, off_v, ends_v, i_v, r_v, pos1_v, pos2_v, rows_v, gout_v,
          sem):
        wid = _sc_wid()
        base = wid * TW
        pltpu.sync_copy(cnt_hbm, cnt_v)
        c = cnt_v[...]
        used = (c + (B_ROWS - 1)) >> 8
        ends = plsc.cumsum(used)
        off = (ends - used) * B_ROWS
        off_v[...] = off
        ends_v[...] = ends

        for (ih, rh, pv) in ((i1_hbm, r1_hbm, pos1_v),
                             (i2_hbm, r2_hbm, pos2_v)):
            pltpu.sync_copy(ih.at[pl.ds(base, TW)], i_v)
            pltpu.sync_copy(rh.at[pl.ds(base, TW)], r_v)
            for j in range(TW // 16):
                sl = pl.ds(j * 16, 16)
                idx = i_v[sl]
                pv[sl] = plsc.load_gather(off_v, [idx]) + r_v[sl]

        pltpu.sync_copy(x_hbm.at[pl.ds(base, TW)], rows_v)
        pltpu.async_copy(rows_v, xs_hbm.at[pos1_v], sem).wait()
        pltpu.async_copy(rows_v, xs_hbm.at[pos2_v], sem).wait()
        pltpu.sync_copy(pos1_v, pos1_hbm.at[pl.ds(base, TW)])
        pltpu.sync_copy(pos2_v, pos2_hbm.at[pl.ds(base, TW)])

        @pl.when(wid == 0)
        def _():
            iota = lax.iota(jnp.int32, 16)
            ends2 = ends_v[...]
            used2 = cnt_v[...]
            usedb = (used2 + (B_ROWS - 1)) >> 8
            nb_used = lax.reduce_max(ends2, (0,))
            last_e = lax.reduce_max(
                jnp.where(usedb > 0, iota, -1), (0,))
            gb0 = jnp.zeros((16,), jnp.int32)
            gb1 = jnp.zeros((16,), jnp.int32)
            b0 = iota
            b1 = iota + 16
            for e in range(NUM_EXPERTS):
                ends_e = lax.reduce_max(
                    jnp.where(iota == e, ends2, -1), (0,))
                gb0 += (b0 >= ends_e).astype(jnp.int32)
                gb1 += (b1 >= ends_e).astype(jnp.int32)
            gb0 = jnp.where(b0 < nb_used, jnp.minimum(gb0, NUM_EXPERTS - 1),
                            last_e)
            gb1 = jnp.where(b1 < nb_used, jnp.minimum(gb1, NUM_EXPERTS - 1),
                            last_e)
            gb1 = jnp.where(iota == 8, nb_used, gb1)
            gout_v[pl.ds(0, 16)] = gb0
            gout_v[pl.ds(16, 16)] = gb1
            pltpu.sync_copy(gout_v, gout_hbm)

    return k(x, i1, i2, r1, r2, cnt)


def _combine_sc(ys, pos1, pos2, g1, g2):
    mesh = plsc.VectorSubcoreMesh(core_axis_name="c", subcore_axis_name="s", num_cores=SC_CORES, num_subcores=SC_SUBCORES)

    @functools.partial(
        pl.kernel,
        out_type=jax.ShapeDtypeStruct((TOKENS, DIM), jnp.float32),
        mesh=mesh,
        compiler_params=pltpu.CompilerParams(needs_layout_passes=False),
        scratch_types=[
            [pltpu.VMEM((HALF,), jnp.int32) for _ in range(2)],
            [pltpu.VMEM((HALF,), jnp.int32) for _ in range(2)],
            pltpu.VMEM((TW,), jnp.float32),
            pltpu.VMEM((TW,), jnp.float32),
            pltpu.VMEM((HALF, DIM), jnp.float32),
            pltpu.VMEM((HALF, DIM), jnp.float32),
            pltpu.SemaphoreType.DMA,
            pltpu.SemaphoreType.DMA,
        ],
    )
    def k(ys_hbm, pos1_hbm, pos2_hbm, g1_hbm, g2_hbm, out_hbm,
          pos1_vs, pos2_vs, g1_v, g2_v, rows1_v, rows2_v, sem1, sem2):
        wid = _sc_wid()
        base = wid * TW
        pltpu.sync_copy(g1_hbm.at[pl.ds(base, TW)], g1_v)
        pltpu.sync_copy(g2_hbm.at[pl.ds(base, TW)], g2_v)
        for h in range(2):
            pltpu.sync_copy(pos1_hbm.at[pl.ds(base + h * HALF, HALF)],
                            pos1_vs[h])
            pltpu.sync_copy(pos2_hbm.at[pl.ds(base + h * HALF, HALF)],
                            pos2_vs[h])
        for h in range(2):
            cp1 = pltpu.async_copy(ys_hbm.at[pos1_vs[h]], rows1_v, sem1)
            cp2 = pltpu.async_copy(ys_hbm.at[pos2_vs[h]], rows2_v, sem2)
            cp1.wait()
            cp2.wait()

            @plsc.parallel_loop(0, HALF)
            def _(t):
                tidx = jnp.zeros((16,), jnp.int32) + (h * HALF + t)
                ga = plsc.load_gather(g1_v, [tidx])
                gb = plsc.load_gather(g2_v, [tidx])

                @plsc.parallel_loop(0, DIM // 16, unroll=8)
                def _(cc):
                    sl = pl.ds(cc * 16, 16)
                    rows1_v[t, sl] = rows1_v[t, sl] * ga + rows2_v[t, sl] * gb
            pltpu.sync_copy(rows1_v,
                            out_hbm.at[pl.ds(base + h * HALF, HALF)])

    return k(ys, pos1, pos2, g1, g2)


def kernel(x, Wr, W1, W2, W3):
    B, S, D = x.shape
    xf = x.reshape(-1, D)

    i1, i2, g1, g2, r1, r2, cnt = _route(xf, Wr)
    xs, pos1, pos2, gout = _dispatch_sc(xf, i1, i2, r1, r2, cnt)
    ys = _ffn(gout, xs, W1, W3, W2)
    out = _combine_sc(ys, pos1, pos2, g1, g2)
    return out.reshape(B, S, D)
